# baseline (speedup 1.0000x reference)
import jax
import jax.numpy as jnp
from jax.experimental import pallas as pl
from jax.experimental.pallas import tpu as pltpu

N = 4096
BLK = 256
GRID = N // BLK

_VIO, _AX, _B, _RCV, _QX, _ATY, _C, _QUAD, _LIN, _VIOT, _RCC = range(11)


def _kkt_kernel(cons_ref,
                Q_ref, A_ref, AT_ref, b_ref, c_ref, x_ref, y_ref,
                Iy_ref, il_ref, iu_ref, l_ref, u_ref,
                vscale_ref, cscale_ref,
                out_ref, acc_ref):
    i = pl.program_id(0)
    relu = jax.nn.relu
    cs = cons_ref[0]

    xun = x_ref[...] / vscale_ref[...] * cs
    yun = y_ref[...] / cscale_ref[...] * cs

    Qx = jnp.dot(Q_ref[...], xun, preferred_element_type=jnp.float32)
    Ax = jnp.dot(A_ref[...], xun, preferred_element_type=jnp.float32)
    ATy = jnp.dot(AT_ref[...], yun, preferred_element_type=jnp.float32)

    sl = pl.ds(i * BLK, BLK)
    xb = x_ref[sl, :] / vscale_ref[sl, :] * cs
    yb = y_ref[sl, :] / cscale_ref[sl, :] * cs
    b = b_ref[...]
    c = c_ref[...]
    il = il_ref[...]
    iu = iu_ref[...]
    Iy = Iy_ref[...]
    l = l_ref[...]
    u = u_ref[...]

    var_vio = relu(l - xb) * il + relu(xb - u) * iu
    cons_vio = b - Ax
    cons_vio = cons_vio + relu(-cons_vio) * Iy

    pg = c - ATy + Qx
    rpg = relu(pg)
    rng = relu(-pg)
    RCV = pg - rpg * il + rng * iu
    DR = relu(-yb) * Iy
    RC = rpg * il - rng * iu
    tm = jnp.where(RC > 0, l, u)

    p_vio = jnp.maximum(jnp.max(jnp.abs(var_vio)), jnp.max(jnp.abs(cons_vio)))
    p_ax = jnp.max(jnp.abs(Ax))
    p_b = jnp.max(jnp.abs(b))
    p_rcv = jnp.maximum(jnp.max(jnp.abs(RCV)), jnp.max(jnp.abs(DR)))
    p_qx = jnp.max(jnp.abs(Qx))
    p_aty = jnp.max(jnp.abs(ATy))
    p_c = jnp.max(jnp.abs(c))
    s_quad = jnp.sum(xb * Qx)
    s_lin = jnp.sum(c * xb)
    s_vio = jnp.sum(b * yb)
    s_rc = jnp.sum(RC * tm)

    @pl.when(i == 0)
    def _init():
        for k in range(11):
            acc_ref[k] = 0.0

    acc_ref[_VIO] = jnp.maximum(acc_ref[_VIO], p_vio)
    acc_ref[_AX] = jnp.maximum(acc_ref[_AX], p_ax)
    acc_ref[_B] = jnp.maximum(acc_ref[_B], p_b)
    acc_ref[_RCV] = jnp.maximum(acc_ref[_RCV], p_rcv)
    acc_ref[_QX] = jnp.maximum(acc_ref[_QX], p_qx)
    acc_ref[_ATY] = jnp.maximum(acc_ref[_ATY], p_aty)
    acc_ref[_C] = jnp.maximum(acc_ref[_C], p_c)
    acc_ref[_QUAD] = acc_ref[_QUAD] + s_quad
    acc_ref[_LIN] = acc_ref[_LIN] + s_lin
    acc_ref[_VIOT] = acc_ref[_VIOT] + s_vio
    acc_ref[_RCC] = acc_ref[_RCC] + s_rc

    @pl.when(i == GRID - 1)
    def _finalize():
        t1 = acc_ref[_VIO] / (1.0 + jnp.maximum(acc_ref[_AX], acc_ref[_B]))
        t2 = acc_ref[_RCV] / (1.0 + jnp.maximum(
            acc_ref[_QX], jnp.maximum(acc_ref[_ATY], acc_ref[_C])))
        quad = acc_ref[_QUAD]
        lin = acc_ref[_LIN]
        vio = acc_ref[_VIOT]
        rcc = acc_ref[_RCC]
        t3 = jnp.abs(quad + lin - vio - rcc) / (
            1.0 + jnp.maximum(jnp.abs(vio - 0.5 * quad),
                              jnp.abs(0.5 * quad + lin)))
        res = jnp.maximum(t1, jnp.maximum(t2, t3))
        out_ref[0] = res
        out_ref[1] = t1
        out_ref[2] = t2
        out_ref[3] = t3


def kernel(Q, A, AT, b, c, x, y, Iy, il, iu, l, u, vscale, cscale, cons_scale):
    b2 = b.reshape(N, 1)
    c2 = c.reshape(N, 1)
    cs = cons_scale.reshape(1)

    row_blk = pl.BlockSpec((BLK, N), lambda i: (i, 0))
    vec_blk = pl.BlockSpec((BLK, 1), lambda i: (i, 0))
    full_vec = pl.BlockSpec((N, 1), lambda i: (0, 0))

    out = pl.pallas_call(
        _kkt_kernel,
        grid=(GRID,),
        in_specs=[
            pl.BlockSpec(memory_space=pltpu.SMEM),
            row_blk,
            row_blk,
            row_blk,
            vec_blk,
            vec_blk,
            full_vec,
            full_vec,
            vec_blk,
            vec_blk,
            vec_blk,
            vec_blk,
            vec_blk,
            full_vec,
            full_vec,
        ],
        out_specs=pl.BlockSpec(memory_space=pltpu.SMEM),
        out_shape=jax.ShapeDtypeStruct((4,), jnp.float32),
        scratch_shapes=[pltpu.SMEM((11,), jnp.float32)],
    )(cs, Q, A, AT, b2, c2, x, y, Iy, il, iu, l, u, vscale, cscale)

    res = out[0].reshape(1, 1)
    t1 = out[1].reshape(())
    t2 = out[2].reshape(())
    t3 = out[3].reshape(1, 1)
    return res, t1, t2, t3
